# baseline (speedup 1.0000x reference)
import math

import jax
import jax.numpy as jnp
from jax.experimental import pallas as pl

_N = 10000
_C = 128
_H = 1
_EPS = 1e-5
_LAYERS = 2


def _gcn(x, src, dst, W, b, n):
    loop = jnp.arange(n, dtype=src.dtype)
    s2 = jnp.concatenate([src, loop])
    d2 = jnp.concatenate([dst, loop])
    deg = jnp.zeros((n,), x.dtype).at[d2].add(1.0)
    dinv = jax.lax.rsqrt(jnp.maximum(deg, 1.0))
    norm = dinv[s2] * dinv[d2]
    h = x @ W
    out = jnp.zeros((n, h.shape[1]), x.dtype).at[d2].add(h[s2] * norm[:, None])
    return out + b


def _edge_softmax(alpha, dst, n):
    amax = jnp.full((n, alpha.shape[1]), -jnp.inf, alpha.dtype).at[dst].max(alpha)
    a = jnp.exp(alpha - amax[dst])
    den = jnp.zeros((n, alpha.shape[1]), alpha.dtype).at[dst].add(a)
    return a / (den[dst] + 1e-16)


def _infusion(x, cond, src, dst, p, n):
    q = (x @ p['Wq'] + p['bq']).reshape(n, _H, _C)
    k = (cond @ p['Wk'] + p['bk']).reshape(n, _H, _C)
    v = (cond @ p['Wv'] + p['bv']).reshape(n, _H, _C)
    alpha = (q[dst] * k[src]).sum(-1) / math.sqrt(_C)
    alpha = _edge_softmax(alpha, dst, n)
    msg = v[src] * alpha[:, :, None]
    out = jnp.zeros((n, _H, _C), x.dtype).at[dst].add(msg)
    out = out.mean(axis=1)
    return out + x @ p['Wskip'] + p['bskip']


def _ln(x, g, b):
    m = x.mean(-1, keepdims=True)
    v = ((x - m) ** 2).mean(-1, keepdims=True)
    return (x - m) / jnp.sqrt(v + 1e-5) * g + b


def _residual_add_kernel(a_ref, b_ref, o_ref):
    o_ref[...] = a_ref[...] + b_ref[...]


def _residual_add(a, b):
    return pl.pallas_call(
        _residual_add_kernel,
        out_shape=jax.ShapeDtypeStruct(a.shape, a.dtype),
        grid=(10,),
        in_specs=[pl.BlockSpec((1000, _C), lambda i: (i, 0)),
                  pl.BlockSpec((1000, _C), lambda i: (i, 0))],
        out_specs=pl.BlockSpec((1000, _C), lambda i: (i, 0)),
    )(a, b)


def kernel(x, edge_index, cond, params):
    src, dst = edge_index[0], edge_index[1]
    n = x.shape[0]
    x_in = x
    m = x.mean(0)
    v = x.var(0)
    h = (x - m) / jnp.sqrt(v + _EPS) * params['bn_g'] + params['bn_b']
    h = _gcn(h, src, dst, params['proj_in_W'], params['proj_in_b'], n)
    bp = params['block']
    for _ in range(_LAYERS):
        h = _infusion(h, cond, src, dst, bp['l1'], n)
        h = _ln(h, bp['ln1_g'], bp['ln1_b'])
        h = _infusion(h, cond, src, dst, bp['l2'], n)
        h = _ln(h, bp['ln2_g'], bp['ln2_b'])
        h = _gcn(h, src, dst, bp['gcn1_W'], bp['gcn1_b'], n)
        h = jax.nn.silu(h)
        h = _gcn(h, src, dst, bp['gcn2_W'], bp['gcn2_b'], n)
    h = _gcn(h, src, dst, params['proj_out_W'], params['proj_out_b'], n)
    return _residual_add(h, x_in)


# speedup vs baseline: 3.2712x; 3.2712x over previous
import functools
import math

import jax
import jax.numpy as jnp
from jax import lax
from jax.experimental import pallas as pl
from jax.experimental.pallas import tpu as pltpu
from jax.experimental.pallas import tpu_sc as plsc

N = 10000
E = 160000
C = 128
DC = 768
EPS = 1e-5
N_LAYERS = 2

NC = 2
NS = 16
NW = NC * NS
CH = 128
E_PAD = 163840
EPW = E_PAD // NW
ITERS = EPW // CH
N_PAD = 10240
RPT = N_PAD // NS

_mesh = plsc.VectorSubcoreMesh(core_axis_name="c", subcore_axis_name="s")
_f32 = jnp.float32


def _wid():
    return lax.axis_index("s") * NC + lax.axis_index("c")


@functools.partial(
    pl.kernel,
    out_type=jax.ShapeDtypeStruct((NC, N_PAD, C), _f32),
    mesh=_mesh,
    scratch_types=[
        pltpu.VMEM((CH,), jnp.int32),
        pltpu.VMEM((CH,), jnp.int32),
        pltpu.VMEM((CH, C), _f32),
        pltpu.VMEM_SHARED((N_PAD, C), _f32),
        pltpu.SemaphoreType.DMA,
    ],
)
def _sc_gcn_pass(hp, srcp, dstp, zrows, s_out, idxs_v, idxd_v, rows_v, acc, sem):
    cid = lax.axis_index("c")
    sid = lax.axis_index("s")
    wid = _wid()
    pltpu.sync_copy(zrows, acc.at[pl.ds(sid * RPT, RPT)])
    plsc.subcore_barrier()

    def body(i, carry):
        off = pl.multiple_of(wid * EPW + i * CH, 8)
        pltpu.sync_copy(srcp.at[pl.ds(off, CH)], idxs_v)
        pltpu.sync_copy(dstp.at[pl.ds(off, CH)], idxd_v)
        pltpu.async_copy(hp.at[idxs_v], rows_v, sem).wait()
        pltpu.sync_copy(rows_v, acc.at[idxd_v], add=True)
        return carry

    lax.fori_loop(0, ITERS, body, 0)
    plsc.subcore_barrier()
    pltpu.sync_copy(acc.at[pl.ds(sid * RPT, RPT)],
                    s_out.at[cid, pl.ds(sid * RPT, RPT)])


@functools.partial(
    pl.kernel,
    out_type=[jax.ShapeDtypeStruct((E_PAD, C), _f32)] * 3,
    mesh=_mesh,
    scratch_types=[
        pltpu.VMEM((CH,), jnp.int32),
        pltpu.VMEM((CH,), jnp.int32),
        pltpu.VMEM((CH, C), _f32),
        pltpu.VMEM((CH, C), _f32),
        pltpu.VMEM((CH, C), _f32),
        pltpu.SemaphoreType.DMA,
    ],
)
def _sc_gather3(qh, kh, vh, dstp, srcp, qe, ke, ve,
                idxd_v, idxs_v, bq, bk, bv, sem):
    wid = _wid()

    def body(i, carry):
        off = pl.multiple_of(wid * EPW + i * CH, 8)
        pltpu.sync_copy(dstp.at[pl.ds(off, CH)], idxd_v)
        pltpu.sync_copy(srcp.at[pl.ds(off, CH)], idxs_v)
        cq = pltpu.async_copy(qh.at[idxd_v], bq, sem)
        ck = pltpu.async_copy(kh.at[idxs_v], bk, sem)
        cv = pltpu.async_copy(vh.at[idxs_v], bv, sem)
        cq.wait()
        ck.wait()
        cv.wait()
        pltpu.sync_copy(bq, qe.at[pl.ds(off, CH)])
        pltpu.sync_copy(bk, ke.at[pl.ds(off, CH)])
        pltpu.sync_copy(bv, ve.at[pl.ds(off, CH)])
        return carry

    lax.fori_loop(0, ITERS, body, 0)


@functools.partial(
    pl.kernel,
    out_type=jax.ShapeDtypeStruct((NC, N_PAD, C), _f32),
    mesh=_mesh,
    scratch_types=[
        pltpu.VMEM((CH,), jnp.int32),
        pltpu.VMEM((CH, C), _f32),
        pltpu.VMEM_SHARED((N_PAD, C), _f32),
    ],
)
def _sc_scatter128(msg, dstp, zrows, s_out, idx_v, rows_v, accm):
    cid = lax.axis_index("c")
    sid = lax.axis_index("s")
    wid = _wid()
    pltpu.sync_copy(zrows, accm.at[pl.ds(sid * RPT, RPT)])
    plsc.subcore_barrier()

    def body(i, carry):
        off = pl.multiple_of(wid * EPW + i * CH, 8)
        pltpu.sync_copy(dstp.at[pl.ds(off, CH)], idx_v)
        pltpu.sync_copy(msg.at[pl.ds(off, CH)], rows_v)
        pltpu.sync_copy(rows_v, accm.at[idx_v], add=True)
        return carry

    lax.fori_loop(0, ITERS, body, 0)
    plsc.subcore_barrier()
    pltpu.sync_copy(accm.at[pl.ds(sid * RPT, RPT)],
                    s_out.at[cid, pl.ds(sid * RPT, RPT)])


_BLK = 1000
_NB = N // _BLK
_EBLK = 1024
_ENB = E_PAD // _EBLK


def _row_spec(blk, width):
    return pl.BlockSpec((blk, width), lambda i: (i, 0))


def _full_spec(shape):
    return pl.BlockSpec(shape, lambda i: (0, 0))


def _tc_bn(x, g, b):
    def kern(x_ref, g_ref, b_ref, o_ref):
        xv = x_ref[...]
        m = jnp.mean(xv, axis=0, keepdims=True)
        v = jnp.mean(xv * xv, axis=0, keepdims=True) - m * m
        o_ref[...] = (xv - m) * (g_ref[...] * lax.rsqrt(v + EPS)) + b_ref[...]

    return pl.pallas_call(
        kern,
        out_shape=jax.ShapeDtypeStruct((N, C), _f32),
    )(x, g, b)


def _tc_dinv(d0, d1):
    def kern(a_ref, b_ref, o_ref):
        o_ref[...] = lax.rsqrt(a_ref[...] + b_ref[...] + 1.0)

    return pl.pallas_call(
        kern,
        out_shape=jax.ShapeDtypeStruct((N, 16), _f32),
    )(d0, d1)


def _tc_matmul(x, w, b):
    kin = x.shape[1]

    def kern(x_ref, w_ref, b_ref, o_ref):
        o_ref[...] = jnp.dot(x_ref[...], w_ref[...],
                             preferred_element_type=_f32) + b_ref[...]

    return pl.pallas_call(
        kern,
        grid=(_NB,),
        in_specs=[_row_spec(_BLK, kin), _full_spec((kin, C)), _full_spec((1, C))],
        out_specs=_row_spec(_BLK, C),
        out_shape=jax.ShapeDtypeStruct((N, C), _f32),
    )(x, w, b)


def _tc_matmul_rs(x, w, rs16):
    def kern(x_ref, w_ref, r_ref, o_ref):
        o_ref[...] = jnp.dot(x_ref[...], w_ref[...],
                             preferred_element_type=_f32) * r_ref[:, 0:1]

    return pl.pallas_call(
        kern,
        grid=(_NB,),
        in_specs=[_row_spec(_BLK, C), _full_spec((C, C)), _row_spec(_BLK, 16)],
        out_specs=_row_spec(_BLK, C),
        out_shape=jax.ShapeDtypeStruct((N, C), _f32),
    )(x, w, rs16)


def _tc_gcn_post(s0, s1, hp, rs16, b, res=None, act=False):
    def kern(*refs):
        if res is None:
            s0_r, s1_r, hp_r, r_r, b_r, o_r = refs
        else:
            s0_r, s1_r, hp_r, r_r, b_r, res_r, o_r = refs
        o = (s0_r[...] + s1_r[...] + hp_r[...]) * r_r[:, 0:1] + b_r[...]
        if act:
            o = o * jax.nn.sigmoid(o)
        if res is not None:
            o = o + res_r[...]
        o_r[...] = o

    ins = [s0, s1, hp, rs16, b] + ([] if res is None else [res])
    specs = [_row_spec(_BLK, C), _row_spec(_BLK, C), _row_spec(_BLK, C),
             _row_spec(_BLK, 16), _full_spec((1, C))]
    if res is not None:
        specs.append(_row_spec(_BLK, C))
    return pl.pallas_call(
        kern,
        grid=(_NB,),
        in_specs=specs,
        out_specs=_row_spec(_BLK, C),
        out_shape=jax.ShapeDtypeStruct((N, C), _f32),
    )(*ins)


def _tc_edge(qe, ke, ve):
    inv = 1.0 / math.sqrt(C)

    def kern(q_ref, k_ref, v_ref, m_ref, e_ref):
        a = jnp.sum(q_ref[...] * k_ref[...], axis=1, keepdims=True) * inv
        e = jnp.exp(a)
        m_ref[...] = v_ref[...] * e
        e_ref[...] = jnp.broadcast_to(e, (_EBLK, C))

    return pl.pallas_call(
        kern,
        grid=(_ENB,),
        in_specs=[_row_spec(_EBLK, C)] * 3,
        out_specs=[_row_spec(_EBLK, C), _row_spec(_EBLK, C)],
        out_shape=[jax.ShapeDtypeStruct((E_PAD, C), _f32),
                   jax.ShapeDtypeStruct((E_PAD, C), _f32)],
    )(qe, ke, ve)


def _tc_attn_post_ln(s0, s1, d0, d1, x, w, b, g, lb):
    def kern(s0_r, s1_r, d0_r, d1_r, x_r, w_r, b_r, g_r, lb_r, o_r):
        den = d0_r[:, 0:1] + d1_r[:, 0:1] + 1e-16
        t = (s0_r[...] + s1_r[...]) / den
        t = t + jnp.dot(x_r[...], w_r[...], preferred_element_type=_f32)
        t = t + b_r[...]
        m = jnp.mean(t, axis=1, keepdims=True)
        v = jnp.mean((t - m) * (t - m), axis=1, keepdims=True)
        o_r[...] = (t - m) * lax.rsqrt(v + 1e-5) * g_r[...] + lb_r[...]

    return pl.pallas_call(
        kern,
        grid=(_NB,),
        in_specs=[_row_spec(_BLK, C), _row_spec(_BLK, C),
                  _row_spec(_BLK, 16), _row_spec(_BLK, 16),
                  _row_spec(_BLK, C), _full_spec((C, C)),
                  _full_spec((1, C)), _full_spec((1, C)), _full_spec((1, C))],
        out_specs=_row_spec(_BLK, C),
        out_shape=jax.ShapeDtypeStruct((N, C), _f32),
    )(s0, s1, d0, d1, x, w, b, g, lb)


def kernel(x, edge_index, cond, params):
    src = edge_index[0]
    dst = edge_index[1]
    pad = E_PAD - E
    srcp = jnp.concatenate([src, jnp.zeros((pad,), jnp.int32)])
    dstp = jnp.concatenate([dst, jnp.full((pad,), N_PAD - 1, jnp.int32)])
    zrows = jnp.zeros((RPT, C), _f32)

    def r1(v):
        return v.reshape(1, C)

    degp = _sc_gcn_pass(jnp.ones((N, C), _f32), srcp, dstp, zrows)
    dinv16 = _tc_dinv(degp[0, :N, :16], degp[1, :N, :16])

    h = _tc_bn(x, r1(params['bn_g']), r1(params['bn_b']))

    def gcn(h, w, b, res=None, act=False):
        hp = _tc_matmul_rs(h, w, dinv16)
        sp = _sc_gcn_pass(hp, srcp, dstp, zrows)
        return _tc_gcn_post(sp[0, :N], sp[1, :N], hp, dinv16, r1(b),
                            res=res, act=act)

    h = gcn(h, params['proj_in_W'], params['proj_in_b'])

    bp = params['block']
    kv = {}
    for l in ('l1', 'l2'):
        p = bp[l]
        kv[l] = (_tc_matmul(cond, p['Wk'], r1(p['bk'])),
                 _tc_matmul(cond, p['Wv'], r1(p['bv'])))

    def infusion_ln(h, p, kh, vh, g, lb):
        q = _tc_matmul(h, p['Wq'], r1(p['bq']))
        qe, ke, ve = _sc_gather3(q, kh, vh, dstp, srcp)
        msg, e128 = _tc_edge(qe, ke, ve)
        sp = _sc_scatter128(msg, dstp, zrows)
        dp = _sc_scatter128(e128, dstp, zrows)
        return _tc_attn_post_ln(sp[0, :N], sp[1, :N],
                                dp[0, :N, :16], dp[1, :N, :16],
                                h, p['Wskip'], r1(p['bskip']), r1(g), r1(lb))

    for _ in range(N_LAYERS):
        h = infusion_ln(h, bp['l1'], kv['l1'][0], kv['l1'][1],
                        bp['ln1_g'], bp['ln1_b'])
        h = infusion_ln(h, bp['l2'], kv['l2'][0], kv['l2'][1],
                        bp['ln2_g'], bp['ln2_b'])
        h = gcn(h, bp['gcn1_W'], bp['gcn1_b'], act=True)
        h = gcn(h, bp['gcn2_W'], bp['gcn2_b'])

    return gcn(h, params['proj_out_W'], params['proj_out_b'], res=x)


# speedup vs baseline: 3.3112x; 1.0122x over previous
import functools
import math

import jax
import jax.numpy as jnp
from jax import lax
from jax.experimental import pallas as pl
from jax.experimental.pallas import tpu as pltpu
from jax.experimental.pallas import tpu_sc as plsc

N = 10000
E = 160000
C = 128
DC = 768
EPS = 1e-5
N_LAYERS = 2

NC = 2
NS = 16
NW = NC * NS
CH = 128
E_PAD = 163840
EPW = E_PAD // NW
ITERS = EPW // CH
N_PAD = 10240
RPT = N_PAD // NS

_mesh = plsc.VectorSubcoreMesh(core_axis_name="c", subcore_axis_name="s")
_f32 = jnp.float32


def _wid():
    return lax.axis_index("s") * NC + lax.axis_index("c")


@functools.partial(
    pl.kernel,
    out_type=jax.ShapeDtypeStruct((NC, N_PAD, C), _f32),
    mesh=_mesh,
    scratch_types=[
        pltpu.VMEM((CH,), jnp.int32),
        pltpu.VMEM((CH,), jnp.int32),
        pltpu.VMEM((CH,), jnp.int32),
        pltpu.VMEM((CH,), jnp.int32),
        pltpu.VMEM((CH, C), _f32),
        pltpu.VMEM((CH, C), _f32),
        pltpu.VMEM_SHARED((N_PAD, C), _f32),
        pltpu.SemaphoreType.DMA,
        pltpu.SemaphoreType.DMA,
    ],
)
def _sc_gcn_pass(hp, srcp, dstp, zrows, s_out, idxs0, idxd0, idxs1, idxd1,
                 rows0, rows1, acc, sem0, sem1):
    cid = lax.axis_index("c")
    sid = lax.axis_index("s")
    wid = _wid()
    pltpu.sync_copy(zrows, acc.at[pl.ds(sid * RPT, RPT)])
    plsc.subcore_barrier()

    def body(i, carry):
        off0 = pl.multiple_of(wid * EPW + i * (2 * CH), 8)
        off1 = pl.multiple_of(wid * EPW + i * (2 * CH) + CH, 8)
        pltpu.sync_copy(srcp.at[pl.ds(off0, CH)], idxs0)
        pltpu.sync_copy(dstp.at[pl.ds(off0, CH)], idxd0)
        pltpu.sync_copy(srcp.at[pl.ds(off1, CH)], idxs1)
        pltpu.sync_copy(dstp.at[pl.ds(off1, CH)], idxd1)
        c0 = pltpu.async_copy(hp.at[idxs0], rows0, sem0)
        c1 = pltpu.async_copy(hp.at[idxs1], rows1, sem1)
        c0.wait()
        pltpu.sync_copy(rows0, acc.at[idxd0], add=True)
        c1.wait()
        pltpu.sync_copy(rows1, acc.at[idxd1], add=True)
        return carry

    lax.fori_loop(0, ITERS // 2, body, 0)
    plsc.subcore_barrier()
    pltpu.sync_copy(acc.at[pl.ds(sid * RPT, RPT)],
                    s_out.at[cid, pl.ds(sid * RPT, RPT)])


@functools.partial(
    pl.kernel,
    out_type=[jax.ShapeDtypeStruct((E_PAD, C), _f32)] * 3,
    mesh=_mesh,
    scratch_types=[
        pltpu.VMEM((CH,), jnp.int32),
        pltpu.VMEM((CH,), jnp.int32),
        pltpu.VMEM((CH, C), _f32),
        pltpu.VMEM((CH, C), _f32),
        pltpu.VMEM((CH, C), _f32),
        pltpu.SemaphoreType.DMA,
    ],
)
def _sc_gather3(qh, kh, vh, dstp, srcp, qe, ke, ve,
                idxd_v, idxs_v, bq, bk, bv, sem):
    wid = _wid()

    def body(i, carry):
        off = pl.multiple_of(wid * EPW + i * CH, 8)
        pltpu.sync_copy(dstp.at[pl.ds(off, CH)], idxd_v)
        pltpu.sync_copy(srcp.at[pl.ds(off, CH)], idxs_v)
        cq = pltpu.async_copy(qh.at[idxd_v], bq, sem)
        ck = pltpu.async_copy(kh.at[idxs_v], bk, sem)
        cv = pltpu.async_copy(vh.at[idxs_v], bv, sem)
        cq.wait()
        ck.wait()
        cv.wait()
        pltpu.sync_copy(bq, qe.at[pl.ds(off, CH)])
        pltpu.sync_copy(bk, ke.at[pl.ds(off, CH)])
        pltpu.sync_copy(bv, ve.at[pl.ds(off, CH)])
        return carry

    lax.fori_loop(0, ITERS, body, 0)


@functools.partial(
    pl.kernel,
    out_type=jax.ShapeDtypeStruct((NC, N_PAD, C), _f32),
    mesh=_mesh,
    scratch_types=[
        pltpu.VMEM((CH,), jnp.int32),
        pltpu.VMEM((CH,), jnp.int32),
        pltpu.VMEM((CH, C), _f32),
        pltpu.VMEM((CH, C), _f32),
        pltpu.VMEM_SHARED((N_PAD, C), _f32),
        pltpu.SemaphoreType.DMA,
        pltpu.SemaphoreType.DMA,
    ],
)
def _sc_scatter128(msg, dstp, zrows, s_out, idx0, idx1, rows0, rows1, accm,
                   sem0, sem1):
    cid = lax.axis_index("c")
    sid = lax.axis_index("s")
    wid = _wid()
    pltpu.sync_copy(zrows, accm.at[pl.ds(sid * RPT, RPT)])
    plsc.subcore_barrier()

    def body(i, carry):
        off0 = pl.multiple_of(wid * EPW + i * (2 * CH), 8)
        off1 = pl.multiple_of(wid * EPW + i * (2 * CH) + CH, 8)
        pltpu.sync_copy(dstp.at[pl.ds(off0, CH)], idx0)
        pltpu.sync_copy(dstp.at[pl.ds(off1, CH)], idx1)
        c0 = pltpu.async_copy(msg.at[pl.ds(off0, CH)], rows0, sem0)
        c1 = pltpu.async_copy(msg.at[pl.ds(off1, CH)], rows1, sem1)
        c0.wait()
        pltpu.sync_copy(rows0, accm.at[idx0], add=True)
        c1.wait()
        pltpu.sync_copy(rows1, accm.at[idx1], add=True)
        return carry

    lax.fori_loop(0, ITERS // 2, body, 0)
    plsc.subcore_barrier()
    pltpu.sync_copy(accm.at[pl.ds(sid * RPT, RPT)],
                    s_out.at[cid, pl.ds(sid * RPT, RPT)])


_BLK = 1000
_NB = N // _BLK
_EBLK = 1024
_ENB = E_PAD // _EBLK


def _row_spec(blk, width):
    return pl.BlockSpec((blk, width), lambda i: (i, 0))


def _full_spec(shape):
    return pl.BlockSpec(shape, lambda i: (0, 0))


def _tc_bn(x, g, b):
    def kern(x_ref, g_ref, b_ref, o_ref):
        xv = x_ref[...]
        m = jnp.mean(xv, axis=0, keepdims=True)
        v = jnp.mean(xv * xv, axis=0, keepdims=True) - m * m
        o_ref[...] = (xv - m) * (g_ref[...] * lax.rsqrt(v + EPS)) + b_ref[...]

    return pl.pallas_call(
        kern,
        out_shape=jax.ShapeDtypeStruct((N, C), _f32),
    )(x, g, b)


def _tc_dinv(d0, d1):
    def kern(a_ref, b_ref, o_ref):
        o_ref[...] = lax.rsqrt(a_ref[...] + b_ref[...] + 1.0)

    return pl.pallas_call(
        kern,
        out_shape=jax.ShapeDtypeStruct((N, 16), _f32),
    )(d0, d1)


def _tc_matmul(x, w, b):
    kin = x.shape[1]

    def kern(x_ref, w_ref, b_ref, o_ref):
        o_ref[...] = jnp.dot(x_ref[...], w_ref[...],
                             preferred_element_type=_f32) + b_ref[...]

    return pl.pallas_call(
        kern,
        grid=(_NB,),
        in_specs=[_row_spec(_BLK, kin), _full_spec((kin, C)), _full_spec((1, C))],
        out_specs=_row_spec(_BLK, C),
        out_shape=jax.ShapeDtypeStruct((N, C), _f32),
    )(x, w, b)


def _tc_matmul_rs(x, w, rs16):
    def kern(x_ref, w_ref, r_ref, o_ref):
        o_ref[...] = jnp.dot(x_ref[...], w_ref[...],
                             preferred_element_type=_f32) * r_ref[:, 0:1]

    return pl.pallas_call(
        kern,
        grid=(_NB,),
        in_specs=[_row_spec(_BLK, C), _full_spec((C, C)), _row_spec(_BLK, 16)],
        out_specs=_row_spec(_BLK, C),
        out_shape=jax.ShapeDtypeStruct((N, C), _f32),
    )(x, w, rs16)


def _tc_gcn_post(s0, s1, hp, rs16, b, res=None, act=False):
    def kern(*refs):
        if res is None:
            s0_r, s1_r, hp_r, r_r, b_r, o_r = refs
        else:
            s0_r, s1_r, hp_r, r_r, b_r, res_r, o_r = refs
        o = (s0_r[...] + s1_r[...] + hp_r[...]) * r_r[:, 0:1] + b_r[...]
        if act:
            o = o * jax.nn.sigmoid(o)
        if res is not None:
            o = o + res_r[...]
        o_r[...] = o

    ins = [s0, s1, hp, rs16, b] + ([] if res is None else [res])
    specs = [_row_spec(_BLK, C), _row_spec(_BLK, C), _row_spec(_BLK, C),
             _row_spec(_BLK, 16), _full_spec((1, C))]
    if res is not None:
        specs.append(_row_spec(_BLK, C))
    return pl.pallas_call(
        kern,
        grid=(_NB,),
        in_specs=specs,
        out_specs=_row_spec(_BLK, C),
        out_shape=jax.ShapeDtypeStruct((N, C), _f32),
    )(*ins)


def _tc_edge(qe, ke, ve):
    inv = 1.0 / math.sqrt(C)

    def kern(q_ref, k_ref, v_ref, m_ref, e_ref):
        a = jnp.sum(q_ref[...] * k_ref[...], axis=1, keepdims=True) * inv
        e = jnp.exp(a)
        m_ref[...] = v_ref[...] * e
        e_ref[...] = jnp.broadcast_to(e, (_EBLK, C))

    return pl.pallas_call(
        kern,
        grid=(_ENB,),
        in_specs=[_row_spec(_EBLK, C)] * 3,
        out_specs=[_row_spec(_EBLK, C), _row_spec(_EBLK, C)],
        out_shape=[jax.ShapeDtypeStruct((E_PAD, C), _f32),
                   jax.ShapeDtypeStruct((E_PAD, C), _f32)],
    )(qe, ke, ve)


def _tc_attn_post_ln(s0, s1, d0, d1, x, w, b, g, lb):
    def kern(s0_r, s1_r, d0_r, d1_r, x_r, w_r, b_r, g_r, lb_r, o_r):
        den = d0_r[:, 0:1] + d1_r[:, 0:1] + 1e-16
        t = (s0_r[...] + s1_r[...]) / den
        t = t + jnp.dot(x_r[...], w_r[...], preferred_element_type=_f32)
        t = t + b_r[...]
        m = jnp.mean(t, axis=1, keepdims=True)
        v = jnp.mean((t - m) * (t - m), axis=1, keepdims=True)
        o_r[...] = (t - m) * lax.rsqrt(v + 1e-5) * g_r[...] + lb_r[...]

    return pl.pallas_call(
        kern,
        grid=(_NB,),
        in_specs=[_row_spec(_BLK, C), _row_spec(_BLK, C),
                  _row_spec(_BLK, 16), _row_spec(_BLK, 16),
                  _row_spec(_BLK, C), _full_spec((C, C)),
                  _full_spec((1, C)), _full_spec((1, C)), _full_spec((1, C))],
        out_specs=_row_spec(_BLK, C),
        out_shape=jax.ShapeDtypeStruct((N, C), _f32),
    )(s0, s1, d0, d1, x, w, b, g, lb)


def kernel(x, edge_index, cond, params):
    src = edge_index[0]
    dst = edge_index[1]
    pad = E_PAD - E
    srcp = jnp.concatenate([src, jnp.zeros((pad,), jnp.int32)])
    dstp = jnp.concatenate([dst, jnp.full((pad,), N_PAD - 1, jnp.int32)])
    zrows = jnp.zeros((RPT, C), _f32)

    def r1(v):
        return v.reshape(1, C)

    degp = _sc_gcn_pass(jnp.ones((N, C), _f32), srcp, dstp, zrows)
    dinv16 = _tc_dinv(degp[0, :N, :16], degp[1, :N, :16])

    h = _tc_bn(x, r1(params['bn_g']), r1(params['bn_b']))

    def gcn(h, w, b, res=None, act=False):
        hp = _tc_matmul_rs(h, w, dinv16)
        sp = _sc_gcn_pass(hp, srcp, dstp, zrows)
        return _tc_gcn_post(sp[0, :N], sp[1, :N], hp, dinv16, r1(b),
                            res=res, act=act)

    h = gcn(h, params['proj_in_W'], params['proj_in_b'])

    bp = params['block']
    kv = {}
    for l in ('l1', 'l2'):
        p = bp[l]
        kv[l] = (_tc_matmul(cond, p['Wk'], r1(p['bk'])),
                 _tc_matmul(cond, p['Wv'], r1(p['bv'])))

    def infusion_ln(h, p, kh, vh, g, lb):
        q = _tc_matmul(h, p['Wq'], r1(p['bq']))
        qe, ke, ve = _sc_gather3(q, kh, vh, dstp, srcp)
        msg, e128 = _tc_edge(qe, ke, ve)
        sp = _sc_scatter128(msg, dstp, zrows)
        dp = _sc_scatter128(e128, dstp, zrows)
        return _tc_attn_post_ln(sp[0, :N], sp[1, :N],
                                dp[0, :N, :16], dp[1, :N, :16],
                                h, p['Wskip'], r1(p['bskip']), r1(g), r1(lb))

    for _ in range(N_LAYERS):
        h = infusion_ln(h, bp['l1'], kv['l1'][0], kv['l1'][1],
                        bp['ln1_g'], bp['ln1_b'])
        h = infusion_ln(h, bp['l2'], kv['l2'][0], kv['l2'][1],
                        bp['ln2_g'], bp['ln2_b'])
        h = gcn(h, bp['gcn1_W'], bp['gcn1_b'], act=True)
        h = gcn(h, bp['gcn2_W'], bp['gcn2_b'])

    return gcn(h, params['proj_out_W'], params['proj_out_b'], res=x)
